# baseline (speedup 1.0000x reference)
import functools

import jax
import jax.numpy as jnp
from jax import lax
from jax.experimental import pallas as pl
from jax.experimental.pallas import tpu as pltpu
from jax.experimental.pallas import tpu_sc as plsc

_NC = 2
_NS = 16
_NW = _NC * _NS
_CHUNK = 128


@functools.lru_cache(maxsize=None)
def _make_sc_agg(npad, f, steps, with_deg):
  rows_per_tile = npad // _NS
  mesh = plsc.VectorSubcoreMesh(core_axis_name="c", subcore_axis_name="s")

  out_type = [jax.ShapeDtypeStruct((_NC, npad, f), jnp.float32)]
  if with_deg:
    out_type.append(jax.ShapeDtypeStruct((_NC * npad,), jnp.float32))

  scratch = [
      pltpu.VMEM((steps, _CHUNK), jnp.int32),
      pltpu.VMEM((steps, _CHUNK), jnp.int32),
      pltpu.VMEM((_CHUNK, f), jnp.float32),
      pltpu.VMEM_SHARED((npad, f), jnp.float32),
  ]
  if with_deg:
    scratch += [
        pltpu.VMEM((_CHUNK,), jnp.float32),
        pltpu.VMEM_SHARED((npad,), jnp.float32),
        pltpu.VMEM((rows_per_tile,), jnp.float32),
    ]

  def body(table, srcm, dstm, zrow, *rest):
    if with_deg:
      (zdeg, out, deg_out, src_v, dst_v, rows_a, acc, ones_v,
       dacc, dbuf) = rest
    else:
      out, src_v, dst_v, rows_a, acc = rest
    c = lax.axis_index("c")
    s = lax.axis_index("s")
    wid = c * _NS + s
    r0 = s * rows_per_tile

    pltpu.sync_copy(zrow.at[pl.ds(r0, rows_per_tile)],
                    acc.at[pl.ds(r0, rows_per_tile)])
    if with_deg:
      pltpu.sync_copy(zdeg.at[pl.ds(r0, rows_per_tile)], dbuf)
      pltpu.sync_copy(dbuf, dacc.at[pl.ds(r0, rows_per_tile)])
      for i in range(_CHUNK // 16):
        ones_v[pl.ds(i * 16, 16)] = jnp.full((16,), 1.0, jnp.float32)

    pltpu.sync_copy(srcm.at[wid], src_v.at[pl.ds(0, steps)])
    pltpu.sync_copy(dstm.at[wid], dst_v)
    plsc.subcore_barrier()

    def step(j, carry):
      pltpu.sync_copy(table.at[src_v.at[j]], rows_a)
      pltpu.sync_copy(rows_a, acc.at[dst_v.at[j]], add=True)
      if with_deg:
        pltpu.sync_copy(ones_v, dacc.at[dst_v.at[j]], add=True)
      return carry

    lax.fori_loop(0, steps, step, 0)
    plsc.subcore_barrier()

    pltpu.sync_copy(acc.at[pl.ds(r0, rows_per_tile)],
                    out.at[c, pl.ds(r0, rows_per_tile)])
    if with_deg:
      pltpu.sync_copy(dacc.at[pl.ds(r0, rows_per_tile)], dbuf)
      pltpu.sync_copy(dbuf, deg_out.at[pl.ds(c * npad + r0, rows_per_tile)])

  return pl.kernel(body, out_type=out_type, mesh=mesh, scratch_types=scratch,
                   compiler_params=pltpu.CompilerParams(
                       use_tc_tiling_on_sc=False))


def _elu(h):
  return jnp.where(h > 0, h, jnp.exp(jnp.minimum(h, 0.0)) - 1.0)


def _bn(s, g, b, n):
  m = jnp.sum(s, axis=0, keepdims=True) / n
  v = jnp.sum(s * s, axis=0, keepdims=True) / n - m * m
  return (s - m) * lax.rsqrt(v + 1e-5) * g + b


def _rowmask(npad, n):
  return lax.broadcasted_iota(jnp.int32, (npad, 1), 0) < n


def _dot(a, b):
  return jnp.dot(a, b, preferred_element_type=jnp.float32)


def _tc_a_body(n, x, w1l, w1r, b1, p1, r1):
  npad = p1.shape[0]
  xv = x[...]
  p1[...] = jnp.concatenate(
      [_dot(xv, w1l[...]),
       jnp.zeros((npad - n, w1l.shape[1]), jnp.float32)], axis=0)
  r1[...] = jnp.concatenate(
      [_dot(xv, w1r[...]) + b1[...],
       jnp.zeros((npad - n, w1r.shape[1]), jnp.float32)], axis=0)


def _tc_b_body(n, agg, deg0, deg1, r1, g1, be1, h1p, dinv):
  npad = r1.shape[0]
  deg = deg0[...] + deg1[...]
  di = 1.0 / jnp.maximum(deg, 1.0)
  mean1 = (agg[0] + agg[1]) * di
  s1 = jnp.where(_rowmask(npad, n), mean1 + r1[...], 0.0)
  h1 = _elu(_bn(s1, g1[...], be1[...], n))
  h1p[...] = jnp.where(_rowmask(npad, n), h1, 0.0)
  dinv[...] = di


def _tc_c_body(n, agg, dinv, h1p, w2l, w2r, b2, g2, be2, w3l, w3r, b3,
               p3p, r3):
  npad = h1p.shape[0]
  mean2 = (agg[0] + agg[1]) * dinv[...]
  s2 = mean2 @ w2l[...] + b2[...] + h1p[...] @ w2r[...]
  s2 = jnp.where(_rowmask(npad, n), s2, 0.0)
  h2 = _elu(_bn(s2, g2[...], be2[...], n))
  h2 = jnp.where(_rowmask(npad, n), h2, 0.0)
  p3p[...] = _dot(h2, w3l[...])
  r3[...] = _dot(h2, w3r[...]) + b3[...]


def _tc_d_body(n, agg, dinv, r3, g3, be3, h1p, w4l, p4p, h4inp):
  npad = r3.shape[0]
  mean3 = (agg[0] + agg[1]) * dinv[...]
  s3 = jnp.where(_rowmask(npad, n), mean3 + r3[...], 0.0)
  h3 = _elu(_bn(s3, g3[...], be3[...], n))
  h3 = jnp.where(_rowmask(npad, n), h3, 0.0)
  h4in = h3 + h1p[...]
  p4p[...] = _dot(h4in, w4l[...])
  h4inp[...] = h4in


def _tc_e_body(n, agg, dinv, h4inp, w4r, b4, g4, be4, wc, bc,
               logits, conv4, bn4):
  npad = h4inp.shape[0]
  mean4 = (agg[0] + agg[1]) * dinv[...]
  c4 = mean4 + b4[...] + _dot(h4inp[...], w4r[...])
  s4 = jnp.where(_rowmask(npad, n), c4, 0.0)
  b4o = _bn(s4, g4[...], be4[...], n)
  h4 = _elu(b4o)
  logits[...] = (_dot(h4, wc[...]) + bc[...])[:n]
  conv4[...] = c4[:n]
  bn4[...] = b4o[:n]


def _tc_call(body, n, out_shapes):
  return pl.pallas_call(
      functools.partial(body, n),
      out_shape=[jax.ShapeDtypeStruct(s, jnp.float32) for s in out_shapes])


def kernel(x, edge_index, W1l, W1r, b1, g1, be1, W2l, W2r, b2, g2, be2,
           W3l, W3r, b3, g3, be3, W4l, W4r, b4, g4, be4, Wc, bc):
  n, gene = x.shape
  e = edge_index.shape[1]
  hid = W1l.shape[1]
  hid2 = W2l.shape[1]
  hid4 = W4l.shape[1]
  ncls = Wc.shape[1]

  rows_per_tile = -(-(n + 1) // (_NS * 8)) * 8
  npad = _NS * rows_per_tile
  steps = -(-e // (_NW * _CHUNK))
  steps += steps % 2
  epad = _NW * _CHUNK * steps

  pad = jnp.full((epad - e,), n, jnp.int32)
  srcm = jnp.concatenate([edge_index[0], pad]).reshape(_NW, steps, _CHUNK)
  dstm = jnp.concatenate([edge_index[1], pad]).reshape(_NW, steps, _CHUNK)

  zrow64 = jnp.zeros((npad, hid), jnp.float32)
  zrow32 = jnp.zeros((npad, hid4), jnp.float32)
  zdeg = jnp.zeros((npad,), jnp.float32)

  agg_deg = _make_sc_agg(npad, hid, steps, True)
  agg64 = _make_sc_agg(npad, hid, steps, False)
  agg32 = _make_sc_agg(npad, hid4, steps, False)

  p1, r1 = _tc_call(_tc_a_body, n, [(npad, hid), (npad, hid)])(
      x, W1l, W1r, b1)
  agg1, degf = agg_deg(p1, srcm, dstm, zrow64, zdeg)
  h1p, dinv = _tc_call(_tc_b_body, n, [(npad, hid), (npad, 1)])(
      agg1, degf[:npad, None], degf[npad:, None], r1, g1, be1)

  [agg2] = agg64(h1p, srcm, dstm, zrow64)
  p3p, r3 = _tc_call(_tc_c_body, n, [(npad, hid), (npad, hid)])(
      agg2, dinv, h1p, W2l, W2r, b2, g2, be2, W3l, W3r, b3)

  [agg3] = agg64(p3p, srcm, dstm, zrow64)
  p4p, h4inp = _tc_call(_tc_d_body, n, [(npad, hid4), (npad, hid)])(
      agg3, dinv, r3, g3, be3, h1p, W4l)

  [agg4] = agg32(p4p, srcm, dstm, zrow32)
  logits, conv4, bn4 = _tc_call(
      _tc_e_body, n, [(n, ncls), (n, hid4), (n, hid4)])(
          agg4, dinv, h4inp, W4r, b4, g4, be4, Wc, bc)

  return (logits, conv4, bn4)


# speedup vs baseline: 1.3861x; 1.3861x over previous
import functools

import jax
import jax.numpy as jnp
from jax import lax
from jax.experimental import pallas as pl
from jax.experimental.pallas import tpu as pltpu
from jax.experimental.pallas import tpu_sc as plsc

_NC = 2
_NS = 16
_NW = _NC * _NS
_CHUNK = 128


@functools.lru_cache(maxsize=None)
def _make_sc_agg(npad, f, steps, with_deg):
  rows_per_tile = npad // _NS
  mesh = plsc.VectorSubcoreMesh(core_axis_name="c", subcore_axis_name="s")

  out_type = [jax.ShapeDtypeStruct((_NC, npad, f), jnp.float32)]
  if with_deg:
    out_type.append(jax.ShapeDtypeStruct((npad,), jnp.float32))
    out_type.append(jax.ShapeDtypeStruct((npad,), jnp.float32))

  scratch = [
      pltpu.VMEM((steps, _CHUNK), jnp.int32),
      pltpu.VMEM((steps, _CHUNK), jnp.int32),
      pltpu.VMEM((_CHUNK, f), jnp.float32),
      pltpu.VMEM_SHARED((npad, f), jnp.float32),
  ]
  if with_deg:
    scratch += [
        pltpu.VMEM((_CHUNK,), jnp.float32),
        pltpu.VMEM_SHARED((npad,), jnp.float32),
        pltpu.VMEM((rows_per_tile,), jnp.float32),
    ]

  def body(table, srcm, dstm, zrow, *rest):
    if with_deg:
      (zdeg, out, deg0_out, deg1_out, src_v, dst_v, rows_a, acc, ones_v,
       dacc, dbuf) = rest
    else:
      out, src_v, dst_v, rows_a, acc = rest
    c = lax.axis_index("c")
    s = lax.axis_index("s")
    wid = c * _NS + s
    r0 = s * rows_per_tile

    pltpu.sync_copy(zrow.at[pl.ds(r0, rows_per_tile)],
                    acc.at[pl.ds(r0, rows_per_tile)])
    if with_deg:
      pltpu.sync_copy(zdeg.at[pl.ds(r0, rows_per_tile)], dbuf)
      pltpu.sync_copy(dbuf, dacc.at[pl.ds(r0, rows_per_tile)])
      for i in range(_CHUNK // 16):
        ones_v[pl.ds(i * 16, 16)] = jnp.full((16,), 1.0, jnp.float32)

    pltpu.sync_copy(srcm.at[wid], src_v)
    pltpu.sync_copy(dstm.at[wid], dst_v)
    plsc.subcore_barrier()

    def step(j, carry):
      pltpu.sync_copy(table.at[src_v.at[j]], rows_a)
      pltpu.sync_copy(rows_a, acc.at[dst_v.at[j]], add=True)
      if with_deg:
        pltpu.sync_copy(ones_v, dacc.at[dst_v.at[j]], add=True)
      return carry

    lax.fori_loop(0, steps, step, 0)
    plsc.subcore_barrier()

    pltpu.sync_copy(acc.at[pl.ds(r0, rows_per_tile)],
                    out.at[c, pl.ds(r0, rows_per_tile)])
    if with_deg:
      pltpu.sync_copy(dacc.at[pl.ds(r0, rows_per_tile)], dbuf)

      @pl.when(c == 0)
      def _():
        pltpu.sync_copy(dbuf, deg0_out.at[pl.ds(r0, rows_per_tile)])

      @pl.when(c == 1)
      def _():
        pltpu.sync_copy(dbuf, deg1_out.at[pl.ds(r0, rows_per_tile)])

  return pl.kernel(body, out_type=out_type, mesh=mesh, scratch_types=scratch,
                   compiler_params=pltpu.CompilerParams(
                       use_tc_tiling_on_sc=False))


def _elu(h):
  return jnp.where(h > 0, h, jnp.exp(jnp.minimum(h, 0.0)) - 1.0)


def _bn(s, g, b, n):
  m = jnp.sum(s, axis=0, keepdims=True) / n
  v = jnp.sum(s * s, axis=0, keepdims=True) / n - m * m
  return (s - m) * lax.rsqrt(v + 1e-5) * g + b


def _rowmask(npad, n):
  return lax.broadcasted_iota(jnp.int32, (npad, 1), 0) < n


def _dot(a, b):
  return jnp.dot(a, b, preferred_element_type=jnp.float32)


def _tc_a_body(n, x, w1l, w1r, b1, p1, r1):
  npad = p1.shape[0]
  xv = x[...]
  p1[...] = jnp.concatenate(
      [_dot(xv, w1l[...]),
       jnp.zeros((npad - n, w1l.shape[1]), jnp.float32)], axis=0)
  r1[...] = jnp.concatenate(
      [_dot(xv, w1r[...]) + b1[...],
       jnp.zeros((npad - n, w1r.shape[1]), jnp.float32)], axis=0)


def _tc_b_body(n, agg, deg0, deg1, r1, g1, be1, h1p, dinv):
  npad = r1.shape[0]
  deg = deg0[...] + deg1[...]
  di = 1.0 / jnp.maximum(deg, 1.0)
  mean1 = (agg[0] + agg[1]) * di
  s1 = jnp.where(_rowmask(npad, n), mean1 + r1[...], 0.0)
  h1 = _elu(_bn(s1, g1[...], be1[...], n))
  h1p[...] = jnp.where(_rowmask(npad, n), h1, 0.0)
  dinv[...] = di


def _tc_c_body(n, agg, dinv, h1p, w2l, w2r, b2, g2, be2, w3l, w3r, b3,
               p3p, r3):
  npad = h1p.shape[0]
  mean2 = (agg[0] + agg[1]) * dinv[...]
  s2 = mean2 @ w2l[...] + b2[...] + h1p[...] @ w2r[...]
  s2 = jnp.where(_rowmask(npad, n), s2, 0.0)
  h2 = _elu(_bn(s2, g2[...], be2[...], n))
  h2 = jnp.where(_rowmask(npad, n), h2, 0.0)
  p3p[...] = _dot(h2, w3l[...])
  r3[...] = _dot(h2, w3r[...]) + b3[...]


def _tc_d_body(n, agg, dinv, r3, g3, be3, h1p, w4l, p4p, h4inp):
  npad = r3.shape[0]
  mean3 = (agg[0] + agg[1]) * dinv[...]
  s3 = jnp.where(_rowmask(npad, n), mean3 + r3[...], 0.0)
  h3 = _elu(_bn(s3, g3[...], be3[...], n))
  h3 = jnp.where(_rowmask(npad, n), h3, 0.0)
  h4in = h3 + h1p[...]
  p4p[...] = _dot(h4in, w4l[...])
  h4inp[...] = h4in


def _tc_e_body(n, agg, dinv, h4inp, w4r, b4, g4, be4, wc, bc,
               logits, conv4, bn4):
  npad = h4inp.shape[0]
  mean4 = (agg[0] + agg[1]) * dinv[...]
  c4 = mean4 + b4[...] + _dot(h4inp[...], w4r[...])
  s4 = jnp.where(_rowmask(npad, n), c4, 0.0)
  b4o = _bn(s4, g4[...], be4[...], n)
  h4 = _elu(b4o)
  logits[...] = (_dot(h4, wc[...]) + bc[...])[:n]
  conv4[...] = c4[:n]
  bn4[...] = b4o[:n]


def _tc_call(body, n, out_shapes):
  return pl.pallas_call(
      functools.partial(body, n),
      out_shape=[jax.ShapeDtypeStruct(s, jnp.float32) for s in out_shapes])


def kernel(x, edge_index, W1l, W1r, b1, g1, be1, W2l, W2r, b2, g2, be2,
           W3l, W3r, b3, g3, be3, W4l, W4r, b4, g4, be4, Wc, bc):
  n, gene = x.shape
  e = edge_index.shape[1]
  hid = W1l.shape[1]
  hid2 = W2l.shape[1]
  hid4 = W4l.shape[1]
  ncls = Wc.shape[1]

  rows_per_tile = -(-(n + 1) // (_NS * 8)) * 8
  npad = _NS * rows_per_tile
  steps = -(-e // (_NW * _CHUNK))
  epad = _NW * _CHUNK * steps

  pad = jnp.full((epad - e,), n, jnp.int32)
  srcm = jnp.concatenate([edge_index[0], pad]).reshape(_NW, steps, _CHUNK)
  dstm = jnp.concatenate([edge_index[1], pad]).reshape(_NW, steps, _CHUNK)

  zrow64 = jnp.zeros((npad, hid), jnp.float32)
  zrow32 = jnp.zeros((npad, hid4), jnp.float32)
  zdeg = jnp.zeros((npad,), jnp.float32)

  agg_deg = _make_sc_agg(npad, hid, steps, True)
  agg64 = _make_sc_agg(npad, hid, steps, False)
  agg32 = _make_sc_agg(npad, hid4, steps, False)

  p1, r1 = _tc_call(_tc_a_body, n, [(npad, hid), (npad, hid)])(
      x, W1l, W1r, b1)
  agg1, deg0, deg1 = agg_deg(p1, srcm, dstm, zrow64, zdeg)
  h1p, dinv = _tc_call(_tc_b_body, n, [(npad, hid), (npad, 1)])(
      agg1, deg0[:, None], deg1[:, None], r1, g1, be1)

  [agg2] = agg64(h1p, srcm, dstm, zrow64)
  p3p, r3 = _tc_call(_tc_c_body, n, [(npad, hid), (npad, hid)])(
      agg2, dinv, h1p, W2l, W2r, b2, g2, be2, W3l, W3r, b3)

  [agg3] = agg64(p3p, srcm, dstm, zrow64)
  p4p, h4inp = _tc_call(_tc_d_body, n, [(npad, hid4), (npad, hid)])(
      agg3, dinv, r3, g3, be3, h1p, W4l)

  [agg4] = agg32(p4p, srcm, dstm, zrow32)
  logits, conv4, bn4 = _tc_call(
      _tc_e_body, n, [(n, ncls), (n, hid4), (n, hid4)])(
          agg4, dinv, h4inp, W4r, b4, g4, be4, Wc, bc)

  return (logits, conv4, bn4)


# speedup vs baseline: 1.9362x; 1.3969x over previous
import functools

import jax
import jax.numpy as jnp
from jax import lax
from jax.experimental import pallas as pl
from jax.experimental.pallas import tpu as pltpu
from jax.experimental.pallas import tpu_sc as plsc

_NC = 2
_NS = 16
_NW = _NC * _NS
_CHUNK = 128


@functools.lru_cache(maxsize=None)
def _make_sc_agg(npad, f, steps, with_deg):
  rows_per_tile = npad // _NS
  mesh = plsc.VectorSubcoreMesh(core_axis_name="c", subcore_axis_name="s")

  out_type = [jax.ShapeDtypeStruct((_NC, npad, f), jnp.float32)]
  if with_deg:
    out_type.append(jax.ShapeDtypeStruct((npad,), jnp.float32))
    out_type.append(jax.ShapeDtypeStruct((npad,), jnp.float32))

  scratch = [
      pltpu.VMEM((steps, _CHUNK), jnp.int32),
      pltpu.VMEM((steps, _CHUNK), jnp.int32),
      pltpu.VMEM((_CHUNK, f), jnp.float32),
      pltpu.VMEM_SHARED((npad, f), jnp.float32),
  ]
  if with_deg:
    scratch += [
        pltpu.VMEM((_CHUNK,), jnp.float32),
        pltpu.VMEM_SHARED((npad,), jnp.float32),
        pltpu.VMEM((rows_per_tile,), jnp.float32),
    ]

  def body(table, srcm, dstm, zrow, *rest):
    if with_deg:
      (zdeg, out, deg0_out, deg1_out, src_v, dst_v, rows_a, acc, ones_v,
       dacc, dbuf) = rest
    else:
      out, src_v, dst_v, rows_a, acc = rest
    c = lax.axis_index("c")
    s = lax.axis_index("s")
    wid = c * _NS + s
    r0 = s * rows_per_tile

    pltpu.sync_copy(zrow.at[pl.ds(r0, rows_per_tile)],
                    acc.at[pl.ds(r0, rows_per_tile)])
    if with_deg:
      pltpu.sync_copy(zdeg.at[pl.ds(r0, rows_per_tile)], dbuf)
      pltpu.sync_copy(dbuf, dacc.at[pl.ds(r0, rows_per_tile)])
      for i in range(_CHUNK // 16):
        ones_v[pl.ds(i * 16, 16)] = jnp.full((16,), 1.0, jnp.float32)

    pltpu.sync_copy(srcm.at[wid], src_v)
    pltpu.sync_copy(dstm.at[wid], dst_v)
    plsc.subcore_barrier()

    def step(j, carry):
      pltpu.sync_copy(table.at[src_v.at[j]], rows_a)
      pltpu.sync_copy(rows_a, acc.at[dst_v.at[j]], add=True)
      if with_deg:
        pltpu.sync_copy(ones_v, dacc.at[dst_v.at[j]], add=True)
      return carry

    lax.fori_loop(0, steps, step, 0)
    plsc.subcore_barrier()

    pltpu.sync_copy(acc.at[pl.ds(r0, rows_per_tile)],
                    out.at[c, pl.ds(r0, rows_per_tile)])
    if with_deg:
      pltpu.sync_copy(dacc.at[pl.ds(r0, rows_per_tile)], dbuf)

      @pl.when(c == 0)
      def _():
        pltpu.sync_copy(dbuf, deg0_out.at[pl.ds(r0, rows_per_tile)])

      @pl.when(c == 1)
      def _():
        pltpu.sync_copy(dbuf, deg1_out.at[pl.ds(r0, rows_per_tile)])

  return pl.kernel(body, out_type=out_type, mesh=mesh, scratch_types=scratch,
                   compiler_params=pltpu.CompilerParams(
                       use_tc_tiling_on_sc=False))


def _elu(h):
  return jnp.where(h > 0, h, jnp.exp(jnp.minimum(h, 0.0)) - 1.0)


def _bn(s, g, b, n):
  m = jnp.sum(s, axis=0, keepdims=True) / n
  v = jnp.sum(s * s, axis=0, keepdims=True) / n - m * m
  return (s - m) * lax.rsqrt(v + 1e-5) * g + b


def _rowmask(npad, n):
  return lax.broadcasted_iota(jnp.int32, (npad, 1), 0) < n


def _dot(a, b):
  return jnp.dot(a, b, preferred_element_type=jnp.float32)


def _tc_a_body(n, x, w1l, w1r, b1, p1, r1):
  npad = p1.shape[0]
  xv = x[...]
  p1[...] = jnp.concatenate(
      [_dot(xv, w1l[...]),
       jnp.zeros((npad - n, w1l.shape[1]), jnp.float32)], axis=0)
  r1[...] = jnp.concatenate(
      [_dot(xv, w1r[...]) + b1[...],
       jnp.zeros((npad - n, w1r.shape[1]), jnp.float32)], axis=0)


def _tc_b_body(n, agg, deg0, deg1, r1, g1, be1, h1p, dinv):
  npad = r1.shape[0]
  deg = deg0[...] + deg1[...]
  di = 1.0 / jnp.maximum(deg, 1.0)
  mean1 = (agg[0] + agg[1]) * di
  s1 = jnp.where(_rowmask(npad, n), mean1 + r1[...], 0.0)
  h1 = _elu(_bn(s1, g1[...], be1[...], n))
  h1p[...] = jnp.where(_rowmask(npad, n), h1, 0.0)
  dinv[...] = di


def _tc_c_body(n, agg, dinv, h1p, w2l, w2r, b2, g2, be2, w3l, w3r, b3,
               p3p, r3):
  npad = h1p.shape[0]
  mean2 = (agg[0] + agg[1]) * dinv[...]
  s2 = mean2 @ w2l[...] + b2[...] + h1p[...] @ w2r[...]
  s2 = jnp.where(_rowmask(npad, n), s2, 0.0)
  h2 = _elu(_bn(s2, g2[...], be2[...], n))
  h2 = jnp.where(_rowmask(npad, n), h2, 0.0)
  p3p[...] = _dot(h2, w3l[...])
  r3[...] = _dot(h2, w3r[...]) + b3[...]


def _tc_d_body(n, agg, dinv, r3, g3, be3, h1p, w4l, p4p, h4inp):
  npad = r3.shape[0]
  mean3 = (agg[0] + agg[1]) * dinv[...]
  s3 = jnp.where(_rowmask(npad, n), mean3 + r3[...], 0.0)
  h3 = _elu(_bn(s3, g3[...], be3[...], n))
  h3 = jnp.where(_rowmask(npad, n), h3, 0.0)
  h4in = h3 + h1p[...]
  p4p[...] = _dot(h4in, w4l[...])
  h4inp[...] = h4in


def _tc_e_body(n, agg, dinv, h4inp, w4r, b4, g4, be4, wc, bc,
               logits, conv4, bn4):
  npad = h4inp.shape[0]
  mean4 = (agg[0] + agg[1]) * dinv[...]
  c4 = mean4 + b4[...] + _dot(h4inp[...], w4r[...])
  s4 = jnp.where(_rowmask(npad, n), c4, 0.0)
  b4o = _bn(s4, g4[...], be4[...], n)
  h4 = _elu(b4o)
  logits[...] = (_dot(h4, wc[...]) + bc[...])[:n]
  conv4[...] = c4[:n]
  bn4[...] = b4o[:n]


def _tc_call(body, n, out_shapes):
  return pl.pallas_call(
      functools.partial(body, n),
      out_shape=[jax.ShapeDtypeStruct(s, jnp.float32) for s in out_shapes])


def kernel(x, edge_index, W1l, W1r, b1, g1, be1, W2l, W2r, b2, g2, be2,
           W3l, W3r, b3, g3, be3, W4l, W4r, b4, g4, be4, Wc, bc):
  n, gene = x.shape
  e = edge_index.shape[1]
  hid = W1l.shape[1]
  hid2 = W2l.shape[1]
  hid4 = W4l.shape[1]
  ncls = Wc.shape[1]

  rows_per_tile = -(-(n + _CHUNK) // (_NS * 8)) * 8
  npad = _NS * rows_per_tile
  steps = -(-e // (_NW * _CHUNK))
  epad = _NW * _CHUNK * steps

  pad = n + jnp.arange(epad - e, dtype=jnp.int32) % (npad - n)
  srcm = jnp.concatenate([edge_index[0], pad]).reshape(_NW, steps, _CHUNK)
  dstm = jnp.concatenate([edge_index[1], pad]).reshape(_NW, steps, _CHUNK)

  zrow64 = jnp.zeros((npad, hid), jnp.float32)
  zrow32 = jnp.zeros((npad, hid4), jnp.float32)
  zdeg = jnp.zeros((npad,), jnp.float32)

  agg_deg = _make_sc_agg(npad, hid, steps, True)
  agg64 = _make_sc_agg(npad, hid, steps, False)
  agg32 = _make_sc_agg(npad, hid4, steps, False)

  p1, r1 = _tc_call(_tc_a_body, n, [(npad, hid), (npad, hid)])(
      x, W1l, W1r, b1)
  agg1, deg0, deg1 = agg_deg(p1, srcm, dstm, zrow64, zdeg)
  h1p, dinv = _tc_call(_tc_b_body, n, [(npad, hid), (npad, 1)])(
      agg1, deg0[:, None], deg1[:, None], r1, g1, be1)

  [agg2] = agg64(h1p, srcm, dstm, zrow64)
  p3p, r3 = _tc_call(_tc_c_body, n, [(npad, hid), (npad, hid)])(
      agg2, dinv, h1p, W2l, W2r, b2, g2, be2, W3l, W3r, b3)

  [agg3] = agg64(p3p, srcm, dstm, zrow64)
  p4p, h4inp = _tc_call(_tc_d_body, n, [(npad, hid4), (npad, hid)])(
      agg3, dinv, r3, g3, be3, h1p, W4l)

  [agg4] = agg32(p4p, srcm, dstm, zrow32)
  logits, conv4, bn4 = _tc_call(
      _tc_e_body, n, [(n, ncls), (n, hid4), (n, hid4)])(
          agg4, dinv, h4inp, W4r, b4, g4, be4, Wc, bc)

  return (logits, conv4, bn4)


# speedup vs baseline: 2.7340x; 1.4120x over previous
import functools

import jax
import jax.numpy as jnp
from jax import lax
from jax.experimental import pallas as pl
from jax.experimental.pallas import tpu as pltpu
from jax.experimental.pallas import tpu_sc as plsc

_NC = 2
_NS = 16
_NW = _NC * _NS
_CHUNK = 128


@functools.lru_cache(maxsize=None)
def _make_sc_agg(npad, f, steps, with_deg):
  rows_per_tile = npad // _NS
  mesh = plsc.VectorSubcoreMesh(core_axis_name="c", subcore_axis_name="s")

  out_type = [jax.ShapeDtypeStruct((_NC, npad, f), jnp.float32)]
  if with_deg:
    out_type.append(jax.ShapeDtypeStruct((_NC * npad,), jnp.float32))

  scratch = [
      pltpu.VMEM((steps + 1, _CHUNK), jnp.int32),
      pltpu.VMEM((steps, _CHUNK), jnp.int32),
      pltpu.VMEM((_CHUNK, f), jnp.float32),
      pltpu.VMEM((_CHUNK, f), jnp.float32),
      pltpu.SemaphoreType.DMA,
      pltpu.SemaphoreType.DMA,
      pltpu.VMEM_SHARED((npad, f), jnp.float32),
  ]
  if with_deg:
    scratch += [
        pltpu.VMEM((_CHUNK,), jnp.float32),
        pltpu.VMEM_SHARED((npad,), jnp.float32),
        pltpu.VMEM((rows_per_tile,), jnp.float32),
    ]

  def body(table, srcm, dstm, zrow, *rest):
    if with_deg:
      (zdeg, out, deg_out, src_v, dst_v, rows_a, rows_b, sem_a, sem_b,
       acc, ones_v, dacc, dbuf) = rest
    else:
      out, src_v, dst_v, rows_a, rows_b, sem_a, sem_b, acc = rest
    c = lax.axis_index("c")
    s = lax.axis_index("s")
    wid = c * _NS + s
    r0 = s * rows_per_tile

    pltpu.sync_copy(zrow.at[pl.ds(r0, rows_per_tile)],
                    acc.at[pl.ds(r0, rows_per_tile)])
    if with_deg:
      pltpu.sync_copy(zdeg.at[pl.ds(r0, rows_per_tile)], dbuf)
      pltpu.sync_copy(dbuf, dacc.at[pl.ds(r0, rows_per_tile)])
      for i in range(_CHUNK // 16):
        ones_v[pl.ds(i * 16, 16)] = jnp.full((16,), 1.0, jnp.float32)

    pltpu.sync_copy(srcm.at[wid], src_v.at[pl.ds(0, steps)])
    pltpu.sync_copy(srcm.at[wid, pl.ds(0, 1)], src_v.at[pl.ds(steps, 1)])
    pltpu.sync_copy(dstm.at[wid], dst_v)
    plsc.subcore_barrier()

    def wait_gather(rows_v, sem):
      pltpu.make_async_copy(table.at[pl.ds(0, _CHUNK)], rows_v, sem).wait()

    def scat(rows_v, j):
      pltpu.sync_copy(rows_v, acc.at[dst_v.at[j]], add=True)
      if with_deg:
        pltpu.sync_copy(ones_v, dacc.at[dst_v.at[j]], add=True)

    pltpu.async_copy(table.at[src_v.at[0]], rows_a, sem_a)

    def step(i, carry):
      j0 = 2 * i
      pltpu.async_copy(table.at[src_v.at[j0 + 1]], rows_b, sem_b)
      wait_gather(rows_a, sem_a)
      scat(rows_a, j0)
      pltpu.async_copy(table.at[src_v.at[j0 + 2]], rows_a, sem_a)
      wait_gather(rows_b, sem_b)
      scat(rows_b, j0 + 1)
      return carry

    lax.fori_loop(0, steps // 2, step, 0)
    wait_gather(rows_a, sem_a)
    plsc.subcore_barrier()

    pltpu.sync_copy(acc.at[pl.ds(r0, rows_per_tile)],
                    out.at[c, pl.ds(r0, rows_per_tile)])
    if with_deg:
      pltpu.sync_copy(dacc.at[pl.ds(r0, rows_per_tile)], dbuf)
      pltpu.sync_copy(dbuf, deg_out.at[pl.ds(c * npad + r0, rows_per_tile)])

  return pl.kernel(body, out_type=out_type, mesh=mesh, scratch_types=scratch,
                   compiler_params=pltpu.CompilerParams(
                       use_tc_tiling_on_sc=False))


def _elu(h):
  return jnp.where(h > 0, h, jnp.exp(jnp.minimum(h, 0.0)) - 1.0)


def _bn(s, g, b, n):
  m = jnp.sum(s, axis=0, keepdims=True) / n
  v = jnp.sum(s * s, axis=0, keepdims=True) / n - m * m
  return (s - m) * lax.rsqrt(v + 1e-5) * g + b


def _rowmask(npad, n):
  return lax.broadcasted_iota(jnp.int32, (npad, 1), 0) < n


def _dot(a, b):
  return jnp.dot(a, b, preferred_element_type=jnp.float32)


def _tc_a_body(n, x, w1l, w1r, b1, p1, r1):
  npad = p1.shape[0]
  xv = x[...]
  p1[...] = jnp.concatenate(
      [_dot(xv, w1l[...]),
       jnp.zeros((npad - n, w1l.shape[1]), jnp.float32)], axis=0)
  r1[...] = jnp.concatenate(
      [_dot(xv, w1r[...]) + b1[...],
       jnp.zeros((npad - n, w1r.shape[1]), jnp.float32)], axis=0)


def _tc_b_body(n, agg, deg0, deg1, r1, g1, be1, h1p, dinv):
  npad = r1.shape[0]
  deg = deg0[...] + deg1[...]
  di = 1.0 / jnp.maximum(deg, 1.0)
  mean1 = (agg[0] + agg[1]) * di
  s1 = jnp.where(_rowmask(npad, n), mean1 + r1[...], 0.0)
  h1 = _elu(_bn(s1, g1[...], be1[...], n))
  h1p[...] = jnp.where(_rowmask(npad, n), h1, 0.0)
  dinv[...] = di


def _tc_c_body(n, agg, dinv, h1p, w2l, w2r, b2, g2, be2, w3l, w3r, b3,
               p3p, r3):
  npad = h1p.shape[0]
  mean2 = (agg[0] + agg[1]) * dinv[...]
  s2 = mean2 @ w2l[...] + b2[...] + h1p[...] @ w2r[...]
  s2 = jnp.where(_rowmask(npad, n), s2, 0.0)
  h2 = _elu(_bn(s2, g2[...], be2[...], n))
  h2 = jnp.where(_rowmask(npad, n), h2, 0.0)
  p3p[...] = _dot(h2, w3l[...])
  r3[...] = _dot(h2, w3r[...]) + b3[...]


def _tc_d_body(n, agg, dinv, r3, g3, be3, h1p, w4l, p4p, h4inp):
  npad = r3.shape[0]
  mean3 = (agg[0] + agg[1]) * dinv[...]
  s3 = jnp.where(_rowmask(npad, n), mean3 + r3[...], 0.0)
  h3 = _elu(_bn(s3, g3[...], be3[...], n))
  h3 = jnp.where(_rowmask(npad, n), h3, 0.0)
  h4in = h3 + h1p[...]
  p4p[...] = _dot(h4in, w4l[...])
  h4inp[...] = h4in


def _tc_e_body(n, agg, dinv, h4inp, w4r, b4, g4, be4, wc, bc,
               logits, conv4, bn4):
  npad = h4inp.shape[0]
  mean4 = (agg[0] + agg[1]) * dinv[...]
  c4 = mean4 + b4[...] + _dot(h4inp[...], w4r[...])
  s4 = jnp.where(_rowmask(npad, n), c4, 0.0)
  b4o = _bn(s4, g4[...], be4[...], n)
  h4 = _elu(b4o)
  logits[...] = (_dot(h4, wc[...]) + bc[...])[:n]
  conv4[...] = c4[:n]
  bn4[...] = b4o[:n]


def _tc_call(body, n, out_shapes):
  return pl.pallas_call(
      functools.partial(body, n),
      out_shape=[jax.ShapeDtypeStruct(s, jnp.float32) for s in out_shapes])


def kernel(x, edge_index, W1l, W1r, b1, g1, be1, W2l, W2r, b2, g2, be2,
           W3l, W3r, b3, g3, be3, W4l, W4r, b4, g4, be4, Wc, bc):
  n, gene = x.shape
  e = edge_index.shape[1]
  hid = W1l.shape[1]
  hid2 = W2l.shape[1]
  hid4 = W4l.shape[1]
  ncls = Wc.shape[1]

  rows_per_tile = -(-(n + _CHUNK) // (_NS * 8)) * 8
  npad = _NS * rows_per_tile
  steps = -(-e // (_NW * _CHUNK))
  steps += steps % 2
  epad = _NW * _CHUNK * steps

  pad = n + jnp.arange(epad - e, dtype=jnp.int32) % (npad - n)
  srcm = jnp.concatenate([edge_index[0], pad]).reshape(_NW, steps, _CHUNK)
  dstm = jnp.concatenate([edge_index[1], pad]).reshape(_NW, steps, _CHUNK)

  zrow64 = jnp.zeros((npad, hid), jnp.float32)
  zrow32 = jnp.zeros((npad, hid4), jnp.float32)
  zdeg = jnp.zeros((npad,), jnp.float32)

  agg_deg = _make_sc_agg(npad, hid, steps, True)
  agg64 = _make_sc_agg(npad, hid, steps, False)
  agg32 = _make_sc_agg(npad, hid4, steps, False)

  p1, r1 = _tc_call(_tc_a_body, n, [(npad, hid), (npad, hid)])(
      x, W1l, W1r, b1)
  agg1, degf = agg_deg(p1, srcm, dstm, zrow64, zdeg)
  h1p, dinv = _tc_call(_tc_b_body, n, [(npad, hid), (npad, 1)])(
      agg1, degf[:npad, None], degf[npad:, None], r1, g1, be1)

  [agg2] = agg64(h1p, srcm, dstm, zrow64)
  p3p, r3 = _tc_call(_tc_c_body, n, [(npad, hid), (npad, hid)])(
      agg2, dinv, h1p, W2l, W2r, b2, g2, be2, W3l, W3r, b3)

  [agg3] = agg64(p3p, srcm, dstm, zrow64)
  p4p, h4inp = _tc_call(_tc_d_body, n, [(npad, hid4), (npad, hid)])(
      agg3, dinv, r3, g3, be3, h1p, W4l)

  [agg4] = agg32(p4p, srcm, dstm, zrow32)
  logits, conv4, bn4 = _tc_call(
      _tc_e_body, n, [(n, ncls), (n, hid4), (n, hid4)])(
          agg4, dinv, h4inp, W4r, b4, g4, be4, Wc, bc)

  return (logits, conv4, bn4)


# speedup vs baseline: 2.9136x; 1.0657x over previous
import functools

import jax
import jax.numpy as jnp
from jax import lax
from jax.experimental import pallas as pl
from jax.experimental.pallas import tpu as pltpu
from jax.experimental.pallas import tpu_sc as plsc

_NC = 2
_NS = 16
_NW = _NC * _NS
_CHUNK = 128


@functools.lru_cache(maxsize=None)
def _make_sc_agg(npad, f, steps, with_deg):
  rows_per_tile = npad // _NS
  mesh = plsc.VectorSubcoreMesh(core_axis_name="c", subcore_axis_name="s")

  out_type = [jax.ShapeDtypeStruct((_NC, npad, f), jnp.float32)]
  if with_deg:
    out_type.append(jax.ShapeDtypeStruct((_NC * npad,), jnp.float32))

  scratch = [
      pltpu.VMEM((steps, _CHUNK), jnp.int32),
      pltpu.VMEM((steps, _CHUNK), jnp.int32),
  ] + [pltpu.VMEM((_CHUNK, f), jnp.float32) for _ in range(4)] + [
      pltpu.SemaphoreType.DMA for _ in range(8)
  ] + [
      pltpu.VMEM_SHARED((npad, f), jnp.float32),
  ]
  if with_deg:
    scratch += [
        pltpu.VMEM((_CHUNK,), jnp.float32),
        pltpu.VMEM_SHARED((npad,), jnp.float32),
        pltpu.VMEM((rows_per_tile,), jnp.float32),
    ]

  def body(table, srcm, dstm, zrow, *rest):
    if with_deg:
      (zdeg, out, deg_out, src_v, dst_v, b0, b1, b2, b3,
       sg0, sg1, sg2, sg3, ss0, ss1, ss2, ss3,
       acc, ones_v, dacc, dbuf) = rest
    else:
      (out, src_v, dst_v, b0, b1, b2, b3,
       sg0, sg1, sg2, sg3, ss0, ss1, ss2, ss3, acc) = rest
    bufs = (b0, b1, b2, b3)
    sgs = (sg0, sg1, sg2, sg3)
    sss = (ss0, ss1, ss2, ss3)
    c = lax.axis_index("c")
    s = lax.axis_index("s")
    wid = c * _NS + s
    r0 = s * rows_per_tile

    pltpu.sync_copy(zrow.at[pl.ds(r0, rows_per_tile)],
                    acc.at[pl.ds(r0, rows_per_tile)])
    if with_deg:
      pltpu.sync_copy(zdeg.at[pl.ds(r0, rows_per_tile)], dbuf)
      pltpu.sync_copy(dbuf, dacc.at[pl.ds(r0, rows_per_tile)])
      for i in range(_CHUNK // 16):
        ones_v[pl.ds(i * 16, 16)] = jnp.full((16,), 1.0, jnp.float32)

    pltpu.sync_copy(srcm.at[wid], src_v)
    pltpu.sync_copy(dstm.at[wid], dst_v)
    plsc.subcore_barrier()

    def g(j, k):
      pltpu.async_copy(table.at[src_v.at[j]], bufs[k], sgs[k])

    def waitg(k):
      pltpu.make_async_copy(table.at[pl.ds(0, _CHUNK)], bufs[k],
                            sgs[k]).wait()

    def scat(j, k):
      pltpu.async_copy(bufs[k], acc.at[dst_v.at[j]], sss[k], add=True)
      if with_deg:
        pltpu.sync_copy(ones_v, dacc.at[dst_v.at[j]], add=True)

    def waits(k):
      pltpu.make_async_copy(bufs[k], acc.at[pl.ds(0, _CHUNK)],
                            sss[k]).wait()

    g(0, 0)
    g(1, 1)
    waitg(0)
    scat(0, 0)
    g(2, 2)
    waitg(1)
    scat(1, 1)
    g(3, 3)

    def trip(t, carry):
      base = 4 * t + 2
      for k2 in range(4):
        j = base + k2
        kb = (2 + k2) % 4
        waitg(kb)
        scat(j, kb)
        waits(k2)
        g(j + 2, k2)
      return carry

    lax.fori_loop(0, (steps - 4) // 4, trip, 0)

    waitg(2)
    scat(steps - 2, 2)
    waitg(3)
    scat(steps - 1, 3)
    for k in range(4):
      waits(k)
    plsc.subcore_barrier()

    pltpu.sync_copy(acc.at[pl.ds(r0, rows_per_tile)],
                    out.at[c, pl.ds(r0, rows_per_tile)])
    if with_deg:
      pltpu.sync_copy(dacc.at[pl.ds(r0, rows_per_tile)], dbuf)
      pltpu.sync_copy(dbuf, deg_out.at[pl.ds(c * npad + r0, rows_per_tile)])

  return pl.kernel(body, out_type=out_type, mesh=mesh, scratch_types=scratch,
                   compiler_params=pltpu.CompilerParams(
                       use_tc_tiling_on_sc=False))


def _elu(h):
  return jnp.where(h > 0, h, jnp.exp(jnp.minimum(h, 0.0)) - 1.0)


def _bn(s, g, b, n):
  m = jnp.sum(s, axis=0, keepdims=True) / n
  v = jnp.sum(s * s, axis=0, keepdims=True) / n - m * m
  return (s - m) * lax.rsqrt(v + 1e-5) * g + b


def _rowmask(npad, n):
  return lax.broadcasted_iota(jnp.int32, (npad, 1), 0) < n


def _dot(a, b):
  return jnp.dot(a, b, preferred_element_type=jnp.float32)


def _tc_a_body(n, x, w1l, w1r, b1, p1, r1):
  npad = p1.shape[0]
  xv = x[...]
  p1[...] = jnp.concatenate(
      [_dot(xv, w1l[...]),
       jnp.zeros((npad - n, w1l.shape[1]), jnp.float32)], axis=0)
  r1[...] = jnp.concatenate(
      [_dot(xv, w1r[...]) + b1[...],
       jnp.zeros((npad - n, w1r.shape[1]), jnp.float32)], axis=0)


def _tc_b_body(n, agg, deg0, deg1, r1, g1, be1, h1p, dinv):
  npad = r1.shape[0]
  deg = deg0[...] + deg1[...]
  di = 1.0 / jnp.maximum(deg, 1.0)
  mean1 = (agg[0] + agg[1]) * di
  s1 = jnp.where(_rowmask(npad, n), mean1 + r1[...], 0.0)
  h1 = _elu(_bn(s1, g1[...], be1[...], n))
  h1p[...] = jnp.where(_rowmask(npad, n), h1, 0.0)
  dinv[...] = di


def _tc_c_body(n, agg, dinv, h1p, w2l, w2r, b2, g2, be2, w3l, w3r, b3,
               p3p, r3):
  npad = h1p.shape[0]
  mean2 = (agg[0] + agg[1]) * dinv[...]
  s2 = mean2 @ w2l[...] + b2[...] + h1p[...] @ w2r[...]
  s2 = jnp.where(_rowmask(npad, n), s2, 0.0)
  h2 = _elu(_bn(s2, g2[...], be2[...], n))
  h2 = jnp.where(_rowmask(npad, n), h2, 0.0)
  p3p[...] = _dot(h2, w3l[...])
  r3[...] = _dot(h2, w3r[...]) + b3[...]


def _tc_d_body(n, agg, dinv, r3, g3, be3, h1p, w4l, p4p, h4inp):
  npad = r3.shape[0]
  mean3 = (agg[0] + agg[1]) * dinv[...]
  s3 = jnp.where(_rowmask(npad, n), mean3 + r3[...], 0.0)
  h3 = _elu(_bn(s3, g3[...], be3[...], n))
  h3 = jnp.where(_rowmask(npad, n), h3, 0.0)
  h4in = h3 + h1p[...]
  p4p[...] = _dot(h4in, w4l[...])
  h4inp[...] = h4in


def _tc_e_body(n, agg, dinv, h4inp, w4r, b4, g4, be4, wc, bc,
               logits, conv4, bn4):
  npad = h4inp.shape[0]
  mean4 = (agg[0] + agg[1]) * dinv[...]
  c4 = mean4 + b4[...] + _dot(h4inp[...], w4r[...])
  s4 = jnp.where(_rowmask(npad, n), c4, 0.0)
  b4o = _bn(s4, g4[...], be4[...], n)
  h4 = _elu(b4o)
  logits[...] = (_dot(h4, wc[...]) + bc[...])[:n]
  conv4[...] = c4[:n]
  bn4[...] = b4o[:n]


def _tc_call(body, n, out_shapes):
  return pl.pallas_call(
      functools.partial(body, n),
      out_shape=[jax.ShapeDtypeStruct(s, jnp.float32) for s in out_shapes])


def kernel(x, edge_index, W1l, W1r, b1, g1, be1, W2l, W2r, b2, g2, be2,
           W3l, W3r, b3, g3, be3, W4l, W4r, b4, g4, be4, Wc, bc):
  n, gene = x.shape
  e = edge_index.shape[1]
  hid = W1l.shape[1]
  hid2 = W2l.shape[1]
  hid4 = W4l.shape[1]
  ncls = Wc.shape[1]

  rows_per_tile = -(-(n + _CHUNK) // (_NS * 8)) * 8
  npad = _NS * rows_per_tile
  steps = -(-e // (_NW * _CHUNK))
  steps += (-steps) % 4
  epad = _NW * _CHUNK * steps

  pad = n + jnp.arange(epad - e, dtype=jnp.int32) % (npad - n)
  srcm = jnp.concatenate([edge_index[0], pad]).reshape(_NW, steps, _CHUNK)
  dstm = jnp.concatenate([edge_index[1], pad]).reshape(_NW, steps, _CHUNK)

  zrow64 = jnp.zeros((npad, hid), jnp.float32)
  zrow32 = jnp.zeros((npad, hid4), jnp.float32)
  zdeg = jnp.zeros((npad,), jnp.float32)

  agg_deg = _make_sc_agg(npad, hid, steps, True)
  agg64 = _make_sc_agg(npad, hid, steps, False)
  agg32 = _make_sc_agg(npad, hid4, steps, False)

  p1, r1 = _tc_call(_tc_a_body, n, [(npad, hid), (npad, hid)])(
      x, W1l, W1r, b1)
  agg1, degf = agg_deg(p1, srcm, dstm, zrow64, zdeg)
  h1p, dinv = _tc_call(_tc_b_body, n, [(npad, hid), (npad, 1)])(
      agg1, degf[:npad, None], degf[npad:, None], r1, g1, be1)

  [agg2] = agg64(h1p, srcm, dstm, zrow64)
  p3p, r3 = _tc_call(_tc_c_body, n, [(npad, hid), (npad, hid)])(
      agg2, dinv, h1p, W2l, W2r, b2, g2, be2, W3l, W3r, b3)

  [agg3] = agg64(p3p, srcm, dstm, zrow64)
  p4p, h4inp = _tc_call(_tc_d_body, n, [(npad, hid4), (npad, hid)])(
      agg3, dinv, r3, g3, be3, h1p, W4l)

  [agg4] = agg32(p4p, srcm, dstm, zrow32)
  logits, conv4, bn4 = _tc_call(
      _tc_e_body, n, [(n, ncls), (n, hid4), (n, hid4)])(
          agg4, dinv, h4inp, W4r, b4, g4, be4, Wc, bc)

  return (logits, conv4, bn4)


# speedup vs baseline: 2.9735x; 1.0205x over previous
import functools

import jax
import jax.numpy as jnp
from jax import lax
from jax.experimental import pallas as pl
from jax.experimental.pallas import tpu as pltpu
from jax.experimental.pallas import tpu_sc as plsc

_NC = 2
_NS = 16
_NW = _NC * _NS
_CHUNK = 128


@functools.lru_cache(maxsize=None)
def _make_sc_agg(npad, f, steps, with_deg):
  rows_per_tile = npad // _NS
  mesh = plsc.VectorSubcoreMesh(core_axis_name="c", subcore_axis_name="s")

  out_type = [jax.ShapeDtypeStruct((_NC, npad, f), jnp.float32)]
  if with_deg:
    out_type.append(jax.ShapeDtypeStruct((_NC * npad,), jnp.float32))

  scratch = [
      pltpu.VMEM((steps, _CHUNK), jnp.int32),
      pltpu.VMEM((steps, _CHUNK), jnp.int32),
  ] + [pltpu.VMEM((_CHUNK, f), jnp.float32) for _ in range(4)] + [
      pltpu.SemaphoreType.DMA for _ in range(8)
  ] + [
      pltpu.VMEM_SHARED((npad, f), jnp.float32),
  ]
  if with_deg:
    scratch += [
        pltpu.VMEM((_CHUNK,), jnp.float32),
        pltpu.VMEM_SHARED((npad,), jnp.float32),
        pltpu.VMEM((rows_per_tile,), jnp.float32),
    ]

  def body(table, srcm, dstm, zrow, *rest):
    if with_deg:
      (zdeg, out, deg_out, src_v, dst_v, b0, b1, b2, b3,
       sg0, sg1, sg2, sg3, ss0, ss1, ss2, ss3,
       acc, ones_v, dacc, dbuf) = rest
    else:
      (out, src_v, dst_v, b0, b1, b2, b3,
       sg0, sg1, sg2, sg3, ss0, ss1, ss2, ss3, acc) = rest
    bufs = (b0, b1, b2, b3)
    sgs = (sg0, sg1, sg2, sg3)
    sss = (ss0, ss1, ss2, ss3)
    c = lax.axis_index("c")
    s = lax.axis_index("s")
    wid = c * _NS + s
    r0 = s * rows_per_tile

    pltpu.async_copy(zrow.at[pl.ds(r0, rows_per_tile)],
                     acc.at[pl.ds(r0, rows_per_tile)], sg0)
    pltpu.async_copy(srcm.at[wid], src_v, sg1)
    pltpu.async_copy(dstm.at[wid], dst_v, sg2)
    if with_deg:
      pltpu.sync_copy(zdeg.at[pl.ds(r0, rows_per_tile)], dbuf)
      pltpu.sync_copy(dbuf, dacc.at[pl.ds(r0, rows_per_tile)])
      for i in range(_CHUNK // 16):
        ones_v[pl.ds(i * 16, 16)] = jnp.full((16,), 1.0, jnp.float32)
    pltpu.make_async_copy(zrow.at[pl.ds(r0, rows_per_tile)],
                          acc.at[pl.ds(r0, rows_per_tile)], sg0).wait()
    pltpu.make_async_copy(srcm.at[wid], src_v, sg1).wait()
    pltpu.make_async_copy(dstm.at[wid], dst_v, sg2).wait()
    plsc.subcore_barrier()

    def g(j, k):
      pltpu.async_copy(table.at[src_v.at[j]], bufs[k], sgs[k])

    def waitg(k):
      pltpu.make_async_copy(table.at[pl.ds(0, _CHUNK)], bufs[k],
                            sgs[k]).wait()

    def scat(j, k):
      pltpu.async_copy(bufs[k], acc.at[dst_v.at[j]], sss[k], add=True)
      if with_deg:
        pltpu.sync_copy(ones_v, dacc.at[dst_v.at[j]], add=True)

    def waits(k):
      pltpu.make_async_copy(bufs[k], acc.at[pl.ds(0, _CHUNK)],
                            sss[k]).wait()

    g(0, 0)
    g(1, 1)
    waitg(0)
    scat(0, 0)
    g(2, 2)
    waitg(1)
    scat(1, 1)
    g(3, 3)

    def trip(t, carry):
      base = 4 * t + 2
      for k2 in range(4):
        j = base + k2
        kb = (2 + k2) % 4
        waitg(kb)
        scat(j, kb)
        waits(k2)
        g(j + 2, k2)
      return carry

    lax.fori_loop(0, (steps - 4) // 4, trip, 0)

    waitg(2)
    scat(steps - 2, 2)
    waitg(3)
    scat(steps - 1, 3)
    for k in range(4):
      waits(k)
    plsc.subcore_barrier()

    pltpu.sync_copy(acc.at[pl.ds(r0, rows_per_tile)],
                    out.at[c, pl.ds(r0, rows_per_tile)])
    if with_deg:
      pltpu.sync_copy(dacc.at[pl.ds(r0, rows_per_tile)], dbuf)
      pltpu.sync_copy(dbuf, deg_out.at[pl.ds(c * npad + r0, rows_per_tile)])

  return pl.kernel(body, out_type=out_type, mesh=mesh, scratch_types=scratch,
                   compiler_params=pltpu.CompilerParams(
                       use_tc_tiling_on_sc=False))


def _elu(h):
  return jnp.where(h > 0, h, jnp.exp(jnp.minimum(h, 0.0)) - 1.0)


def _bn(s, g, b, n):
  m = jnp.sum(s, axis=0, keepdims=True) / n
  v = jnp.sum(s * s, axis=0, keepdims=True) / n - m * m
  return (s - m) * lax.rsqrt(v + 1e-5) * g + b


def _rowmask(npad, n):
  return lax.broadcasted_iota(jnp.int32, (npad, 1), 0) < n


def _dot(a, b):
  return jnp.dot(a, b, preferred_element_type=jnp.float32)


def _tc_a_body(n, x, w1l, w1r, b1, p1, r1):
  npad = p1.shape[0]
  xv = x[...]
  p1[...] = jnp.concatenate(
      [_dot(xv, w1l[...]),
       jnp.zeros((npad - n, w1l.shape[1]), jnp.float32)], axis=0)
  r1[...] = jnp.concatenate(
      [_dot(xv, w1r[...]) + b1[...],
       jnp.zeros((npad - n, w1r.shape[1]), jnp.float32)], axis=0)


def _tc_b_body(n, agg, deg0, deg1, r1, g1, be1, h1p, dinv):
  npad = r1.shape[0]
  deg = deg0[...] + deg1[...]
  di = 1.0 / jnp.maximum(deg, 1.0)
  mean1 = (agg[0] + agg[1]) * di
  s1 = jnp.where(_rowmask(npad, n), mean1 + r1[...], 0.0)
  h1 = _elu(_bn(s1, g1[...], be1[...], n))
  h1p[...] = jnp.where(_rowmask(npad, n), h1, 0.0)
  dinv[...] = di


def _tc_c_body(n, agg, dinv, h1p, w2l, w2r, b2, g2, be2, w3l, w3r, b3,
               p3p, r3):
  npad = h1p.shape[0]
  mean2 = (agg[0] + agg[1]) * dinv[...]
  s2 = mean2 @ w2l[...] + b2[...] + h1p[...] @ w2r[...]
  s2 = jnp.where(_rowmask(npad, n), s2, 0.0)
  h2 = _elu(_bn(s2, g2[...], be2[...], n))
  h2 = jnp.where(_rowmask(npad, n), h2, 0.0)
  p3p[...] = _dot(h2, w3l[...])
  r3[...] = _dot(h2, w3r[...]) + b3[...]


def _tc_d_body(n, agg, dinv, r3, g3, be3, h1p, w4l, p4p, h4inp):
  npad = r3.shape[0]
  mean3 = (agg[0] + agg[1]) * dinv[...]
  s3 = jnp.where(_rowmask(npad, n), mean3 + r3[...], 0.0)
  h3 = _elu(_bn(s3, g3[...], be3[...], n))
  h3 = jnp.where(_rowmask(npad, n), h3, 0.0)
  h4in = h3 + h1p[...]
  p4p[...] = _dot(h4in, w4l[...])
  h4inp[...] = h4in


def _tc_e_body(n, agg, dinv, h4inp, w4r, b4, g4, be4, wc, bc,
               logits, conv4, bn4):
  npad = h4inp.shape[0]
  mean4 = (agg[0] + agg[1]) * dinv[...]
  c4 = mean4 + b4[...] + _dot(h4inp[...], w4r[...])
  s4 = jnp.where(_rowmask(npad, n), c4, 0.0)
  b4o = _bn(s4, g4[...], be4[...], n)
  h4 = _elu(b4o)
  logits[...] = (_dot(h4, wc[...]) + bc[...])[:n]
  conv4[...] = c4[:n]
  bn4[...] = b4o[:n]


def _tc_call(body, n, out_shapes):
  return pl.pallas_call(
      functools.partial(body, n),
      out_shape=[jax.ShapeDtypeStruct(s, jnp.float32) for s in out_shapes])


def kernel(x, edge_index, W1l, W1r, b1, g1, be1, W2l, W2r, b2, g2, be2,
           W3l, W3r, b3, g3, be3, W4l, W4r, b4, g4, be4, Wc, bc):
  n, gene = x.shape
  e = edge_index.shape[1]
  hid = W1l.shape[1]
  hid2 = W2l.shape[1]
  hid4 = W4l.shape[1]
  ncls = Wc.shape[1]

  rows_per_tile = -(-(n + _CHUNK) // (_NS * 8)) * 8
  npad = _NS * rows_per_tile
  steps = -(-e // (_NW * _CHUNK))
  steps += (-steps) % 4
  epad = _NW * _CHUNK * steps

  pad = n + jnp.arange(epad - e, dtype=jnp.int32) % (npad - n)
  srcm = jnp.concatenate([edge_index[0], pad]).reshape(_NW, steps, _CHUNK)
  dstm = jnp.concatenate([edge_index[1], pad]).reshape(_NW, steps, _CHUNK)

  zrow64 = jnp.zeros((npad, hid), jnp.float32)
  zrow32 = jnp.zeros((npad, hid4), jnp.float32)
  zdeg = jnp.zeros((npad,), jnp.float32)

  agg_deg = _make_sc_agg(npad, hid, steps, True)
  agg64 = _make_sc_agg(npad, hid, steps, False)
  agg32 = _make_sc_agg(npad, hid4, steps, False)

  p1, r1 = _tc_call(_tc_a_body, n, [(npad, hid), (npad, hid)])(
      x, W1l, W1r, b1)
  agg1, degf = agg_deg(p1, srcm, dstm, zrow64, zdeg)
  h1p, dinv = _tc_call(_tc_b_body, n, [(npad, hid), (npad, 1)])(
      agg1, degf[:npad, None], degf[npad:, None], r1, g1, be1)

  [agg2] = agg64(h1p, srcm, dstm, zrow64)
  p3p, r3 = _tc_call(_tc_c_body, n, [(npad, hid), (npad, hid)])(
      agg2, dinv, h1p, W2l, W2r, b2, g2, be2, W3l, W3r, b3)

  [agg3] = agg64(p3p, srcm, dstm, zrow64)
  p4p, h4inp = _tc_call(_tc_d_body, n, [(npad, hid4), (npad, hid)])(
      agg3, dinv, r3, g3, be3, h1p, W4l)

  [agg4] = agg32(p4p, srcm, dstm, zrow32)
  logits, conv4, bn4 = _tc_call(
      _tc_e_body, n, [(n, ncls), (n, hid4), (n, hid4)])(
          agg4, dinv, h4inp, W4r, b4, g4, be4, Wc, bc)

  return (logits, conv4, bn4)


# speedup vs baseline: 2.9759x; 1.0008x over previous
import functools

import jax
import jax.numpy as jnp
from jax import lax
from jax.experimental import pallas as pl
from jax.experimental.pallas import tpu as pltpu
from jax.experimental.pallas import tpu_sc as plsc

_NC = 2
_NS = 16
_NW = _NC * _NS
_CHUNK = 128


@functools.lru_cache(maxsize=None)
def _make_sc_agg(npad, f, steps, with_deg):
  rows_per_tile = npad // _NS
  mesh = plsc.VectorSubcoreMesh(core_axis_name="c", subcore_axis_name="s")

  out_type = [jax.ShapeDtypeStruct((_NC, npad, f), jnp.float32)]
  if with_deg:
    out_type.append(jax.ShapeDtypeStruct((_NC * npad,), jnp.float32))

  scratch = [
      pltpu.VMEM((steps, _CHUNK), jnp.int32),
      pltpu.VMEM((steps, _CHUNK), jnp.int32),
  ] + [pltpu.VMEM((_CHUNK, f), jnp.float32) for _ in range(4)] + [
      pltpu.SemaphoreType.DMA for _ in range(8)
  ] + [
      pltpu.VMEM_SHARED((npad, f), jnp.float32),
  ]
  if with_deg:
    scratch += [
        pltpu.VMEM((_CHUNK,), jnp.float32),
        pltpu.VMEM_SHARED((npad,), jnp.float32),
        pltpu.VMEM((rows_per_tile,), jnp.float32),
    ]

  def body(table, srcm, dstm, zrow, *rest):
    if with_deg:
      (zdeg, out, deg_out, src_v, dst_v, b0, b1, b2, b3,
       sg0, sg1, sg2, sg3, ss0, ss1, ss2, ss3,
       acc, ones_v, dacc, dbuf) = rest
    else:
      (out, src_v, dst_v, b0, b1, b2, b3,
       sg0, sg1, sg2, sg3, ss0, ss1, ss2, ss3, acc) = rest
    bufs = (b0, b1, b2, b3)
    sgs = (sg0, sg1, sg2, sg3)
    sss = (ss0, ss1, ss2, ss3)
    c = lax.axis_index("c")
    s = lax.axis_index("s")
    wid = c * _NS + s
    r0 = s * rows_per_tile

    pltpu.async_copy(zrow.at[pl.ds(r0, rows_per_tile)],
                     acc.at[pl.ds(r0, rows_per_tile)], sg0)
    pltpu.async_copy(srcm.at[wid], src_v, sg1)
    pltpu.async_copy(dstm.at[wid], dst_v, sg2)
    if with_deg:
      pltpu.sync_copy(zdeg.at[pl.ds(r0, rows_per_tile)], dbuf)
      pltpu.sync_copy(dbuf, dacc.at[pl.ds(r0, rows_per_tile)])
      for i in range(_CHUNK // 16):
        ones_v[pl.ds(i * 16, 16)] = jnp.full((16,), 1.0, jnp.float32)
    pltpu.make_async_copy(zrow.at[pl.ds(r0, rows_per_tile)],
                          acc.at[pl.ds(r0, rows_per_tile)], sg0).wait()
    pltpu.make_async_copy(srcm.at[wid], src_v, sg1).wait()
    pltpu.make_async_copy(dstm.at[wid], dst_v, sg2).wait()
    plsc.subcore_barrier()

    def g(j, k):
      pltpu.async_copy(table.at[src_v.at[j]], bufs[k], sgs[k])

    def waitg(k):
      pltpu.make_async_copy(table.at[pl.ds(0, _CHUNK)], bufs[k],
                            sgs[k]).wait()

    def scat(j, k):
      pltpu.async_copy(bufs[k], acc.at[dst_v.at[j]], sss[k], add=True)
      if with_deg:
        pltpu.sync_copy(ones_v, dacc.at[dst_v.at[j]], add=True)

    def waits(k):
      pltpu.make_async_copy(bufs[k], acc.at[pl.ds(0, _CHUNK)],
                            sss[k]).wait()

    g(0, 0)
    g(1, 1)
    waitg(0)
    scat(0, 0)
    g(2, 2)
    waitg(1)
    scat(1, 1)
    g(3, 3)

    def trip(t, carry):
      base = 4 * t + 2
      for k2 in range(4):
        j = base + k2
        kb = (2 + k2) % 4
        waitg(kb)
        scat(j, kb)
        waits(k2)
        g(j + 2, k2)
      return carry

    lax.fori_loop(0, (steps - 4) // 4, trip, 0)

    waitg(2)
    scat(steps - 2, 2)
    waitg(3)
    scat(steps - 1, 3)
    for k in range(4):
      waits(k)
    plsc.subcore_barrier()

    pltpu.sync_copy(acc.at[pl.ds(r0, rows_per_tile)],
                    out.at[c, pl.ds(r0, rows_per_tile)])
    if with_deg:
      pltpu.sync_copy(dacc.at[pl.ds(r0, rows_per_tile)], dbuf)
      pltpu.sync_copy(dbuf, deg_out.at[pl.ds(c * npad + r0, rows_per_tile)])

  return pl.kernel(body, out_type=out_type, mesh=mesh, scratch_types=scratch,
                   compiler_params=pltpu.CompilerParams(
                       use_tc_tiling_on_sc=False))


def _elu(h):
  return jnp.where(h > 0, h, jnp.exp(jnp.minimum(h, 0.0)) - 1.0)


def _bn(s, g, b, n):
  m = jnp.sum(s, axis=0, keepdims=True) / n
  v = jnp.sum(s * s, axis=0, keepdims=True) / n - m * m
  return (s - m) * lax.rsqrt(v + 1e-5) * g + b


def _rowmask(npad, n):
  return lax.broadcasted_iota(jnp.int32, (npad, 1), 0) < n


def _dot(a, b):
  return jnp.dot(a, b, preferred_element_type=jnp.float32)


def _tc_a_body(n, x, w1l, w1r, b1, p1, r1):
  npad = p1.shape[0]
  xv = x[...]
  p1[...] = jnp.concatenate(
      [_dot(xv, w1l[...]),
       jnp.zeros((npad - n, w1l.shape[1]), jnp.float32)], axis=0)
  r1[...] = jnp.concatenate(
      [_dot(xv, w1r[...]) + b1[...],
       jnp.zeros((npad - n, w1r.shape[1]), jnp.float32)], axis=0)


def _tc_b_body(n, agg, deg0, deg1, r1, g1, be1, h1p, dinv):
  npad = r1.shape[0]
  deg = deg0[...] + deg1[...]
  di = 1.0 / jnp.maximum(deg, 1.0)
  mean1 = (agg[0] + agg[1]) * di
  s1 = jnp.where(_rowmask(npad, n), mean1 + r1[...], 0.0)
  h1 = _elu(_bn(s1, g1[...], be1[...], n))
  h1p[...] = jnp.where(_rowmask(npad, n), h1, 0.0)
  dinv[...] = di


def _tc_c_body(n, agg, dinv, h1p, w2l, w2r, b2, g2, be2, w3l, w3r, b3,
               p3p, r3):
  npad = h1p.shape[0]
  mean2 = (agg[0] + agg[1]) * dinv[...]
  s2 = mean2 @ w2l[...] + b2[...] + h1p[...] @ w2r[...]
  s2 = jnp.where(_rowmask(npad, n), s2, 0.0)
  h2 = _elu(_bn(s2, g2[...], be2[...], n))
  h2 = jnp.where(_rowmask(npad, n), h2, 0.0)
  p3p[...] = _dot(h2, w3l[...])
  r3[...] = _dot(h2, w3r[...]) + b3[...]


def _tc_d_body(n, agg, dinv, r3, g3, be3, h1p, w4l, p4p, h4inp):
  npad = r3.shape[0]
  mean3 = (agg[0] + agg[1]) * dinv[...]
  s3 = jnp.where(_rowmask(npad, n), mean3 + r3[...], 0.0)
  h3 = _elu(_bn(s3, g3[...], be3[...], n))
  h3 = jnp.where(_rowmask(npad, n), h3, 0.0)
  h4in = h3 + h1p[...]
  p4p[...] = _dot(h4in, w4l[...])
  h4inp[...] = h4in


def _tc_e_body(n, agg, dinv, h4inp, w4r, b4, g4, be4, wc, bc,
               logits, conv4, bn4):
  npad = h4inp.shape[0]
  mean4 = (agg[0] + agg[1]) * dinv[...]
  c4 = mean4 + b4[...] + _dot(h4inp[...], w4r[...])
  s4 = jnp.where(_rowmask(npad, n), c4, 0.0)
  b4o = _bn(s4, g4[...], be4[...], n)
  h4 = _elu(b4o)
  logits[...] = (_dot(h4, wc[...]) + bc[...])[:n]
  conv4[...] = c4[:n]
  bn4[...] = b4o[:n]


def _tc_call(body, n, out_shapes):
  return pl.pallas_call(
      functools.partial(body, n),
      out_shape=[jax.ShapeDtypeStruct(s, jnp.float32) for s in out_shapes])


def kernel(x, edge_index, W1l, W1r, b1, g1, be1, W2l, W2r, b2, g2, be2,
           W3l, W3r, b3, g3, be3, W4l, W4r, b4, g4, be4, Wc, bc):
  n = x.shape[0]
  e = edge_index.shape[1]
  hid = W1l.shape[1]
  hid4 = W4l.shape[1]
  ncls = Wc.shape[1]

  rows_per_tile = -(-(n + _CHUNK) // (_NS * 8)) * 8
  npad = _NS * rows_per_tile
  steps = -(-e // (_NW * _CHUNK))
  steps += (-steps) % 4
  epad = _NW * _CHUNK * steps

  pad = n + jnp.arange(epad - e, dtype=jnp.int32) % (npad - n)
  srcm = jnp.concatenate([edge_index[0], pad]).reshape(_NW, steps, _CHUNK)
  dstm = jnp.concatenate([edge_index[1], pad]).reshape(_NW, steps, _CHUNK)

  zrow64 = jnp.zeros((npad, hid), jnp.float32)
  zrow32 = jnp.zeros((npad, hid4), jnp.float32)
  zdeg = jnp.zeros((npad,), jnp.float32)

  agg_deg = _make_sc_agg(npad, hid, steps, True)
  agg64 = _make_sc_agg(npad, hid, steps, False)
  agg32 = _make_sc_agg(npad, hid4, steps, False)

  p1, r1 = _tc_call(_tc_a_body, n, [(npad, hid), (npad, hid)])(
      x, W1l, W1r, b1)
  agg1, degf = agg_deg(p1, srcm, dstm, zrow64, zdeg)
  h1p, dinv = _tc_call(_tc_b_body, n, [(npad, hid), (npad, 1)])(
      agg1, degf[:npad, None], degf[npad:, None], r1, g1, be1)

  [agg2] = agg64(h1p, srcm, dstm, zrow64)
  p3p, r3 = _tc_call(_tc_c_body, n, [(npad, hid), (npad, hid)])(
      agg2, dinv, h1p, W2l, W2r, b2, g2, be2, W3l, W3r, b3)

  [agg3] = agg64(p3p, srcm, dstm, zrow64)
  p4p, h4inp = _tc_call(_tc_d_body, n, [(npad, hid4), (npad, hid)])(
      agg3, dinv, r3, g3, be3, h1p, W4l)

  [agg4] = agg32(p4p, srcm, dstm, zrow32)
  logits, conv4, bn4 = _tc_call(
      _tc_e_body, n, [(n, ncls), (n, hid4), (n, hid4)])(
          agg4, dinv, h4inp, W4r, b4, g4, be4, Wc, bc)

  return (logits, conv4, bn4)
